# baseline (speedup 1.0000x reference)
import functools

import jax
import jax.numpy as jnp
from jax import lax
from jax.experimental import pallas as pl
from jax.experimental.pallas import tpu as pltpu
from jax.experimental.pallas import tpu_sc as plsc

M_ROWS = 1000000
DIM = 64
BATCH = 16384
LR = 0.01

NC = 2
NS = 16
NW = NC * NS
NW_ACT = 25
ROWS_PER_W = M_ROWS // NW_ACT
BLOCK_ROWS = 8000
NBPT = ROWS_PER_W // BLOCK_ROWS
NB = M_ROWS // BLOCK_ROWS
IDX_PIECE = 2048
NPIECE = BATCH // IDX_PIECE
WAVE = 128
WIN = 512
CAP = BATCH + 1280
TOT = NW * CAP


def _sc_body(idx_hbm, val_hbm, starts_hbm, counts_hbm, rows_hbm, vs_hbm,
             ipiece, hidx, hpos, h2idx, h2pos, cvm, stage, cgat, rgat, vbuf,
             sem_g):
    wid = lax.axis_index("s") * NC + lax.axis_index("c")
    lo = jnp.where(wid < NW_ACT, wid * ROWS_PER_W, 1 << 28)
    hi = lo + ROWS_PER_W
    lanes = lax.iota(jnp.int32, 16)

    def piece_body(p, cnt):
        pltpu.sync_copy(idx_hbm.at[pl.ds(p * IDX_PIECE, IDX_PIECE)], ipiece)

        def scan_body(j, cnt):
            v = ipiece[pl.ds(j * 16, 16)]
            m = (v >= lo) & (v < hi)
            mi = m.astype(jnp.int32)
            dst = cnt + plsc.cumsum(mi) - mi
            plsc.store_scatter(hidx, [dst], v, mask=m)
            pos = p * IDX_PIECE + j * 16 + lanes
            plsc.store_scatter(hpos, [dst], pos, mask=m)
            return cnt + jnp.sum(mi)

        return lax.fori_loop(0, IDX_PIECE // 16, scan_body, cnt)

    cnt = lax.fori_loop(0, NPIECE, piece_body, jnp.int32(0))
    hidx[pl.ds(cnt, 16)] = jnp.full((16,), 1 << 30, jnp.int32)
    nj = cnt // 16 + 1

    k = jnp.int32(0)
    for b in range(NBPT):
        blo = lo + b * BLOCK_ROWS

        def part_body(j, k, blo=blo):
            v = hidx[pl.ds(j * 16, 16)]
            p = hpos[pl.ds(j * 16, 16)]
            m = (v >= blo) & (v < blo + BLOCK_ROWS)
            mi = m.astype(jnp.int32)
            dst = k + plsc.cumsum(mi) - mi
            plsc.store_scatter(h2idx, [dst], v, mask=m)
            plsc.store_scatter(h2pos, [dst], p, mask=m)
            return k + jnp.sum(mi)

        k = lax.fori_loop(0, nj, part_body, k)
        plsc.store_scatter(cvm, [jnp.full((16,), b, jnp.int32)],
                           jnp.full((16,), k, jnp.int32), mask=lanes == 0)
        for g in range(8):
            h2pos[pl.ds(k + g * 16, 16)] = jnp.zeros((16,), jnp.int32)
        k = (k + 127) // 128 * 128

    cum = cvm[pl.ds(0, 16)]
    prev = plsc.load_gather(cvm, [jnp.maximum(lanes - 1, 0)])
    prev = jnp.where(lanes == 0, 0, prev)
    prev = (prev + 127) // 128 * 128
    stage[pl.ds(0, 16)] = wid * CAP + prev
    stage[pl.ds(16, 16)] = cum - prev
    pltpu.sync_copy(stage.at[pl.ds(0, 16)], starts_hbm.at[pl.ds(wid * 16, 16)])
    pltpu.sync_copy(stage.at[pl.ds(16, 16)], counts_hbm.at[pl.ds(wid * 16, 16)])

    kfin = k

    def wave_body(wi, _):
        w0 = wi * WAVE
        for g in range(WAVE // 16):
            off = w0 + g * 16
            rgat[pl.ds(g * 16, 16)] = h2idx[pl.ds(off, 16)]
            cgat[pl.ds(g * 16, 16)] = h2pos[pl.ds(off, 16)]
        pltpu.async_copy(val_hbm.at[cgat], vbuf, sem_g).wait()
        pltpu.sync_copy(rgat, rows_hbm.at[pl.ds(wid * CAP + w0, WAVE)])
        pltpu.sync_copy(vbuf, vs_hbm.at[pl.ds(wid * CAP + w0, WAVE)])
        return 0

    lax.fori_loop(0, kfin // WAVE, wave_body, 0)


@jax.jit
def _sc_stage(idx32, val):
    mesh = plsc.VectorSubcoreMesh(core_axis_name="c", subcore_axis_name="s")
    f = functools.partial(
        pl.kernel,
        out_type=(
            jax.ShapeDtypeStruct((NW * 16,), jnp.int32),
            jax.ShapeDtypeStruct((NW * 16,), jnp.int32),
            jax.ShapeDtypeStruct((TOT,), jnp.int32),
            jax.ShapeDtypeStruct((TOT, DIM), jnp.float32),
        ),
        mesh=mesh,
        compiler_params=pltpu.CompilerParams(
            needs_layout_passes=False, use_tc_tiling_on_sc=False),
        scratch_types=[
            pltpu.VMEM((IDX_PIECE,), jnp.int32),
            pltpu.VMEM((BATCH + 32,), jnp.int32),
            pltpu.VMEM((BATCH + 32,), jnp.int32),
            pltpu.VMEM((BATCH + 1280 + 32,), jnp.int32),
            pltpu.VMEM((BATCH + 1280 + 32,), jnp.int32),
            pltpu.VMEM((16,), jnp.int32),
            pltpu.VMEM((32,), jnp.int32),
            pltpu.VMEM((WAVE,), jnp.int32),
            pltpu.VMEM((WAVE,), jnp.int32),
            pltpu.VMEM((WAVE, DIM), jnp.float32),
            pltpu.SemaphoreType.DMA,
        ],
    )(_sc_body)
    return f(idx32, val)


def _tc_body(starts_sm, counts_sm, mem_any, rows_hbm, vs_hbm, out_any,
             buf, rwin, vwin, semi, semo, sem_r, sem_v):
    b = pl.program_id(0)
    p = lax.rem(b, 2)
    q = 1 - p
    ent = (b // NBPT) * 16 + (b % NBPT)
    start = pl.multiple_of(starts_sm[ent], 128)
    n = counts_sm[ent]
    base = b * BLOCK_ROWS

    def in_copy(blk, par):
        return pltpu.make_async_copy(
            mem_any.at[pl.ds(blk * BLOCK_ROWS, BLOCK_ROWS), :],
            buf.at[par], semi.at[par])

    def out_copy(blk, par):
        return pltpu.make_async_copy(
            buf.at[par],
            out_any.at[pl.ds(blk * BLOCK_ROWS, BLOCK_ROWS), :], semo.at[par])

    @pl.when(b == 0)
    def _():
        in_copy(0, 0).start()

    @pl.when(b >= 1)
    def _():
        out_copy(b - 1, q).wait()

    @pl.when(b + 1 < NB)
    def _():
        in_copy(b + 1, q).start()

    cr = pltpu.make_async_copy(rows_hbm.at[pl.ds(start, WIN)], rwin, sem_r)
    cv = pltpu.make_async_copy(vs_hbm.at[pl.ds(start, WIN)], vwin, sem_v)
    cr.start()
    cv.start()
    in_copy(b, p).wait()
    cr.wait()
    cv.wait()

    def patch_span(done, nw):
        def patch(i, _):
            r = rwin[i] - base
            buf[p, pl.ds(r, 1), :] = (buf[p, pl.ds(r, 1), :]
                                      - LR * vwin[pl.ds(i, 1), :])
            return 0

        lax.fori_loop(0, nw, patch, 0)

    patch_span(0, jnp.minimum(WIN, n))

    def win_body(wi, _):
        done = wi * WIN
        cr = pltpu.make_async_copy(rows_hbm.at[pl.ds(start + done, WIN)],
                                   rwin, sem_r)
        cv = pltpu.make_async_copy(vs_hbm.at[pl.ds(start + done, WIN)],
                                   vwin, sem_v)
        cr.start()
        cv.start()
        cr.wait()
        cv.wait()
        patch_span(done, jnp.minimum(WIN, n - done))
        return 0

    lax.fori_loop(1, (n + WIN - 1) // WIN, win_body, 0)

    out_copy(b, p).start()
    @pl.when(b == NB - 1)
    def _():
        out_copy(b, p).wait()


@jax.jit
def _tc_apply(starts, counts, mem, rows_sorted, val_sorted):
    grid_spec = pltpu.PrefetchScalarGridSpec(
        num_scalar_prefetch=2,
        grid=(NB,),
        in_specs=[
            pl.BlockSpec(memory_space=pl.ANY),
            pl.BlockSpec(memory_space=pl.ANY),
            pl.BlockSpec(memory_space=pl.ANY),
        ],
        out_specs=pl.BlockSpec(memory_space=pl.ANY),
        scratch_shapes=[
            pltpu.VMEM((2, BLOCK_ROWS, DIM), jnp.float32),
            pltpu.SMEM((WIN,), jnp.int32),
            pltpu.VMEM((WIN, DIM), jnp.float32),
            pltpu.SemaphoreType.DMA((2,)),
            pltpu.SemaphoreType.DMA((2,)),
            pltpu.SemaphoreType.DMA,
            pltpu.SemaphoreType.DMA,
        ],
    )
    return pl.pallas_call(
        _tc_body,
        grid_spec=grid_spec,
        out_shape=jax.ShapeDtypeStruct((M_ROWS, DIM), jnp.float32),
        compiler_params=pltpu.CompilerParams(
            dimension_semantics=("arbitrary",)),
    )(starts, counts, mem, rows_sorted, val_sorted)


def kernel(mem, idx, val):
    idx32 = idx.astype(jnp.int32)
    starts, counts, rows_sorted, val_sorted = _sc_stage(idx32, val)
    return _tc_apply(starts, counts, mem, rows_sorted, val_sorted)


# speedup vs baseline: 1.2599x; 1.2599x over previous
import functools

import jax
import jax.numpy as jnp
from jax import lax
from jax.experimental import pallas as pl
from jax.experimental.pallas import tpu as pltpu
from jax.experimental.pallas import tpu_sc as plsc

M_ROWS = 1000000
DIM = 64
BATCH = 16384
LR = 0.01

NC = 2
NS = 16
NW = NC * NS
NW_ACT = 25
ROWS_PER_W = M_ROWS // NW_ACT
BLOCK_ROWS = 8000
NBPT = ROWS_PER_W // BLOCK_ROWS
NB = M_ROWS // BLOCK_ROWS
IDX_PIECE = 2048
NPIECE = BATCH // IDX_PIECE
WAVE = 128
WIN = 512
CAP = BATCH + 1280
TOT = NW * CAP


def _sc_body(idx_hbm, val_hbm, starts_hbm, counts_hbm, rows_hbm, vs_hbm,
             ipiece, hidx, hpos, h2idx, h2pos, cvm, stage, cgat, rgat, vbuf,
             sem_g):
    wid = lax.axis_index("s") * NC + lax.axis_index("c")
    lo = jnp.where(wid < NW_ACT, wid * ROWS_PER_W, 1 << 28)
    hi = lo + ROWS_PER_W
    lanes = lax.iota(jnp.int32, 16)

    def piece_body(p, cnt):
        pltpu.sync_copy(idx_hbm.at[pl.ds(p * IDX_PIECE, IDX_PIECE)], ipiece)

        def scan_body(j, cnt):
            v = ipiece[pl.ds(j * 16, 16)]
            m = (v >= lo) & (v < hi)
            mi = m.astype(jnp.int32)
            dst = cnt + plsc.cumsum(mi) - mi
            plsc.store_scatter(hidx, [dst], v, mask=m)
            pos = p * IDX_PIECE + j * 16 + lanes
            plsc.store_scatter(hpos, [dst], pos, mask=m)
            return cnt + jnp.sum(mi)

        return lax.fori_loop(0, IDX_PIECE // 16, scan_body, cnt)

    cnt = lax.fori_loop(0, NPIECE, piece_body, jnp.int32(0))
    hidx[pl.ds(cnt, 16)] = jnp.full((16,), 1 << 30, jnp.int32)
    nj = cnt // 16 + 1

    k = jnp.int32(0)
    for b in range(NBPT):
        blo = lo + b * BLOCK_ROWS

        def part_body(j, k, blo=blo):
            v = hidx[pl.ds(j * 16, 16)]
            p = hpos[pl.ds(j * 16, 16)]
            m = (v >= blo) & (v < blo + BLOCK_ROWS)
            mi = m.astype(jnp.int32)
            dst = k + plsc.cumsum(mi) - mi
            plsc.store_scatter(h2idx, [dst], v, mask=m)
            plsc.store_scatter(h2pos, [dst], p, mask=m)
            return k + jnp.sum(mi)

        k = lax.fori_loop(0, nj, part_body, k)
        plsc.store_scatter(cvm, [jnp.full((16,), b, jnp.int32)],
                           jnp.full((16,), k, jnp.int32), mask=lanes == 0)
        for g in range(8):
            h2pos[pl.ds(k + g * 16, 16)] = jnp.zeros((16,), jnp.int32)
        k = (k + 127) // 128 * 128

    cum = cvm[pl.ds(0, 16)]
    prev = plsc.load_gather(cvm, [jnp.maximum(lanes - 1, 0)])
    prev = jnp.where(lanes == 0, 0, prev)
    prev = (prev + 127) // 128 * 128
    stage[pl.ds(0, 16)] = wid * CAP + prev
    stage[pl.ds(16, 16)] = cum - prev
    pltpu.sync_copy(stage.at[pl.ds(0, 16)], starts_hbm.at[pl.ds(wid * 16, 16)])
    pltpu.sync_copy(stage.at[pl.ds(16, 16)], counts_hbm.at[pl.ds(wid * 16, 16)])

    kfin = k

    def wave_body(wi, _):
        w0 = wi * WAVE
        for g in range(WAVE // 16):
            off = w0 + g * 16
            rgat[pl.ds(g * 16, 16)] = h2idx[pl.ds(off, 16)]
            cgat[pl.ds(g * 16, 16)] = h2pos[pl.ds(off, 16)]
        pltpu.async_copy(val_hbm.at[cgat], vbuf, sem_g).wait()
        pltpu.sync_copy(rgat, rows_hbm.at[pl.ds(wid * CAP + w0, WAVE)])
        pltpu.sync_copy(vbuf, vs_hbm.at[pl.ds(wid * CAP + w0, WAVE)])
        return 0

    lax.fori_loop(0, kfin // WAVE, wave_body, 0)


@jax.jit
def _sc_stage(idx32, val):
    mesh = plsc.VectorSubcoreMesh(core_axis_name="c", subcore_axis_name="s")
    f = functools.partial(
        pl.kernel,
        out_type=(
            jax.ShapeDtypeStruct((NW * 16,), jnp.int32),
            jax.ShapeDtypeStruct((NW * 16,), jnp.int32),
            jax.ShapeDtypeStruct((TOT,), jnp.int32),
            jax.ShapeDtypeStruct((TOT, DIM), jnp.float32),
        ),
        mesh=mesh,
        compiler_params=pltpu.CompilerParams(
            needs_layout_passes=False, use_tc_tiling_on_sc=False),
        scratch_types=[
            pltpu.VMEM((IDX_PIECE,), jnp.int32),
            pltpu.VMEM((BATCH + 32,), jnp.int32),
            pltpu.VMEM((BATCH + 32,), jnp.int32),
            pltpu.VMEM((BATCH + 1280 + 32,), jnp.int32),
            pltpu.VMEM((BATCH + 1280 + 32,), jnp.int32),
            pltpu.VMEM((16,), jnp.int32),
            pltpu.VMEM((32,), jnp.int32),
            pltpu.VMEM((WAVE,), jnp.int32),
            pltpu.VMEM((WAVE,), jnp.int32),
            pltpu.VMEM((WAVE, DIM), jnp.float32),
            pltpu.SemaphoreType.DMA,
        ],
    )(_sc_body)
    return f(idx32, val)


def _tc_body(starts_sm, counts_sm, mem_ref, rows_hbm, vs_hbm, out_ref,
             rwin, vwin, sem_r, sem_v):
    b = pl.program_id(0)
    out_ref[...] = mem_ref[...]
    ent = (b // NBPT) * 16 + (b % NBPT)
    start = pl.multiple_of(starts_sm[ent], 128)
    n = counts_sm[ent]
    base = b * BLOCK_ROWS

    def win_body(wi, _):
        done = wi * WIN
        cr = pltpu.make_async_copy(rows_hbm.at[pl.ds(start + done, WIN)],
                                   rwin, sem_r)
        cv = pltpu.make_async_copy(vs_hbm.at[pl.ds(start + done, WIN)],
                                   vwin, sem_v)
        cr.start()
        cv.start()
        cr.wait()
        cv.wait()
        nw = jnp.minimum(WIN, n - done)

        def patch(i, _):
            r = rwin[i] - base
            out_ref[0, pl.ds(r, 1), :] = (out_ref[0, pl.ds(r, 1), :]
                                          - LR * vwin[pl.ds(i, 1), :])
            return 0

        lax.fori_loop(0, nw, patch, 0)
        return 0

    lax.fori_loop(0, (n + WIN - 1) // WIN, win_body, 0)


@jax.jit
def _tc_apply(starts, counts, mem, rows_sorted, val_sorted):
    grid_spec = pltpu.PrefetchScalarGridSpec(
        num_scalar_prefetch=2,
        grid=(NB,),
        in_specs=[
            pl.BlockSpec((1, BLOCK_ROWS, DIM), lambda b, s, c: (b, 0, 0)),
            pl.BlockSpec(memory_space=pl.ANY),
            pl.BlockSpec(memory_space=pl.ANY),
        ],
        out_specs=pl.BlockSpec((1, BLOCK_ROWS, DIM),
                               lambda b, s, c: (b, 0, 0)),
        scratch_shapes=[
            pltpu.SMEM((WIN,), jnp.int32),
            pltpu.VMEM((WIN, DIM), jnp.float32),
            pltpu.SemaphoreType.DMA,
            pltpu.SemaphoreType.DMA,
        ],
    )
    out = pl.pallas_call(
        _tc_body,
        grid_spec=grid_spec,
        out_shape=jax.ShapeDtypeStruct((NB, BLOCK_ROWS, DIM), jnp.float32),
        compiler_params=pltpu.CompilerParams(
            dimension_semantics=("arbitrary",)),
    )(starts, counts, mem.reshape(NB, BLOCK_ROWS, DIM),
      rows_sorted, val_sorted)
    return out.reshape(M_ROWS, DIM)


def kernel(mem, idx, val):
    idx32 = idx.astype(jnp.int32)
    starts, counts, rows_sorted, val_sorted = _sc_stage(idx32, val)
    return _tc_apply(starts, counts, mem, rows_sorted, val_sorted)
